# baseline (speedup 1.0000x reference)
import numpy as np
import jax
import jax.numpy as jnp
from jax import lax
from jax.experimental import pallas as pl
from jax.experimental.pallas import tpu as pltpu
from jax.experimental.pallas import tpu_sc as plsc

MAXLEN_ = 2048
D_MODEL_ = 768
BATCH_ = 4
LANES_ = 16

NW_ = 32
ROWS_ = BATCH_ * MAXLEN_
POS_W_ = MAXLEN_ // NW_
CHUNK_ = 32
HALF_ = POS_W_ // CHUNK_
NCHUNK_ = BATCH_ * HALF_
VECS_ = D_MODEL_ // LANES_


def _positional_encoding(maxlen, d_model):
    pos = np.arange(maxlen, dtype=np.float32)[:, None]
    i = np.arange(d_model, dtype=np.float32)[None, :]
    angle_rates = 1.0 / np.power(10000.0, (2.0 * np.floor(i / 2.0)) / np.float32(d_model))
    angle_rads = pos * angle_rates
    pe = np.zeros((maxlen, d_model), dtype=np.float32)
    pe[:, 0::2] = np.sin(angle_rads[:, 0::2])
    pe[:, 1::2] = np.cos(angle_rads[:, 1::2])
    return jnp.asarray(pe)


def _emb_body(table_hbm, idx_hbm, pe_hbm, out_hbm,
              idx_v, rows0, rows1, pe_v, gsem0, gsem1, osem0, osem1):
    wid = lax.axis_index("s") * 2 + lax.axis_index("c")
    pbase = wid * POS_W_
    bufs = (rows0, rows1)
    gsems = (gsem0, gsem1)
    osems = (osem0, osem1)

    pltpu.sync_copy(idx_hbm.at[wid], idx_v)
    gcopy = [None, None]
    ocopy = [None, None]
    gcopy[0] = pltpu.async_copy(table_hbm.at[idx_v.at[0]], rows0, gsem0)
    pltpu.sync_copy(pe_hbm.at[pl.ds(pbase, POS_W_)], pe_v)

    for k in range(NCHUNK_):
        buf = bufs[k % 2]
        gcopy[k % 2].wait()
        if k + 1 < NCHUNK_:
            nb = (k + 1) % 2
            if ocopy[nb] is not None:
                ocopy[nb].wait()
            gcopy[nb] = pltpu.async_copy(
                table_hbm.at[idx_v.at[k + 1]], bufs[nb], gsems[nb])

        h = k % HALF_
        poff = h * CHUNK_

        def add_row(r, carry):
            for j in range(VECS_):
                sl = pl.ds(j * LANES_, LANES_)
                buf[r, sl] = buf[r, sl] + pe_v[poff + r, sl]
            return carry

        lax.fori_loop(0, CHUNK_, add_row, 0)
        b = k // HALF_
        obase = b * MAXLEN_ + pbase + poff
        ocopy[k % 2] = pltpu.async_copy(
            buf, out_hbm.at[pl.ds(obase, CHUNK_)], osems[k % 2])

    ocopy[0].wait()
    ocopy[1].wait()


def kernel(x, token_emb_table):
    idx = (x.reshape(BATCH_, NW_, HALF_, CHUNK_)
             .transpose(1, 0, 2, 3)
             .reshape(NW_, NCHUNK_, CHUNK_)
             .astype(jnp.int32))
    pe = _positional_encoding(MAXLEN_, D_MODEL_)
    mesh = plsc.VectorSubcoreMesh(core_axis_name="c", subcore_axis_name="s")
    out = pl.kernel(
        _emb_body,
        out_type=jax.ShapeDtypeStruct((ROWS_, D_MODEL_), jnp.float32),
        mesh=mesh,
        scratch_types=[
            pltpu.VMEM((NCHUNK_, CHUNK_), jnp.int32),
            pltpu.VMEM((CHUNK_, D_MODEL_), jnp.float32),
            pltpu.VMEM((CHUNK_, D_MODEL_), jnp.float32),
            pltpu.VMEM((POS_W_, D_MODEL_), jnp.float32),
            pltpu.SemaphoreType.DMA,
            pltpu.SemaphoreType.DMA,
            pltpu.SemaphoreType.DMA,
            pltpu.SemaphoreType.DMA,
        ],
    )(token_emb_table, idx, pe)
    return out.reshape(BATCH_, MAXLEN_, D_MODEL_)


# speedup vs baseline: 1.0765x; 1.0765x over previous
import numpy as np
import jax
import jax.numpy as jnp
from jax import lax
from jax.experimental import pallas as pl
from jax.experimental.pallas import tpu as pltpu
from jax.experimental.pallas import tpu_sc as plsc

MAXLEN_ = 2048
D_MODEL_ = 768
BATCH_ = 4
LANES_ = 16

NW_ = 32
ROWS_ = BATCH_ * MAXLEN_
POS_W_ = MAXLEN_ // NW_
CHUNK_ = 32
HALF_ = POS_W_ // CHUNK_
NCHUNK_ = BATCH_ * HALF_
VECS_ = D_MODEL_ // LANES_


def _positional_encoding(maxlen, d_model):
    pos = np.arange(maxlen, dtype=np.float32)[:, None]
    i = np.arange(d_model, dtype=np.float32)[None, :]
    angle_rates = 1.0 / np.power(10000.0, (2.0 * np.floor(i / 2.0)) / np.float32(d_model))
    angle_rads = pos * angle_rates
    pe = np.zeros((maxlen, d_model), dtype=np.float32)
    pe[:, 0::2] = np.sin(angle_rads[:, 0::2])
    pe[:, 1::2] = np.cos(angle_rads[:, 1::2])
    return jnp.asarray(pe)


def _emb_body(table_hbm, idx_hbm, pe_hbm, out_hbm,
              idx_v, rows0, rows1, pe_v, gsem0, gsem1, osem0, osem1):
    wid = lax.axis_index("s") * 2 + lax.axis_index("c")
    pbase = wid * POS_W_
    bufs = (rows0, rows1)
    gsems = (gsem0, gsem1)
    osems = (osem0, osem1)

    pltpu.sync_copy(idx_hbm.at[wid], idx_v)
    gcopy = [None, None]
    ocopy = [None, None]
    gcopy[0] = pltpu.async_copy(table_hbm.at[idx_v.at[0]], rows0, gsem0)
    pltpu.sync_copy(pe_hbm.at[pl.ds(pbase, POS_W_)], pe_v)

    for k in range(NCHUNK_):
        buf = bufs[k % 2]
        gcopy[k % 2].wait()
        if k + 1 < NCHUNK_:
            nb = (k + 1) % 2
            if ocopy[nb] is not None:
                ocopy[nb].wait()
            gcopy[nb] = pltpu.async_copy(
                table_hbm.at[idx_v.at[k + 1]], bufs[nb], gsems[nb])

        h = k % HALF_
        poff = h * CHUNK_

        def add_row(r, carry):
            for j in range(VECS_):
                sl = pl.ds(j * LANES_, LANES_)
                plsc.addupdate(buf.at[r, sl], pe_v[poff + r, sl])
            return carry

        lax.fori_loop(0, CHUNK_, add_row, 0)
        b = k // HALF_
        obase = b * MAXLEN_ + pbase + poff
        ocopy[k % 2] = pltpu.async_copy(
            buf, out_hbm.at[pl.ds(obase, CHUNK_)], osems[k % 2])

    ocopy[0].wait()
    ocopy[1].wait()


def kernel(x, token_emb_table):
    idx = (x.reshape(BATCH_, NW_, HALF_, CHUNK_)
             .transpose(1, 0, 2, 3)
             .reshape(NW_, NCHUNK_, CHUNK_)
             .astype(jnp.int32))
    pe = _positional_encoding(MAXLEN_, D_MODEL_)
    mesh = plsc.VectorSubcoreMesh(core_axis_name="c", subcore_axis_name="s")
    out = pl.kernel(
        _emb_body,
        out_type=jax.ShapeDtypeStruct((ROWS_, D_MODEL_), jnp.float32),
        mesh=mesh,
        scratch_types=[
            pltpu.VMEM((NCHUNK_, CHUNK_), jnp.int32),
            pltpu.VMEM((CHUNK_, D_MODEL_), jnp.float32),
            pltpu.VMEM((CHUNK_, D_MODEL_), jnp.float32),
            pltpu.VMEM((POS_W_, D_MODEL_), jnp.float32),
            pltpu.SemaphoreType.DMA,
            pltpu.SemaphoreType.DMA,
            pltpu.SemaphoreType.DMA,
            pltpu.SemaphoreType.DMA,
        ],
    )(token_emb_table, idx, pe)
    return out.reshape(BATCH_, MAXLEN_, D_MODEL_)


# speedup vs baseline: 1.3746x; 1.2769x over previous
import numpy as np
import jax
import jax.numpy as jnp
from jax import lax
from jax.experimental import pallas as pl
from jax.experimental.pallas import tpu as pltpu
from jax.experimental.pallas import tpu_sc as plsc

MAXLEN_ = 2048
D_MODEL_ = 768
BATCH_ = 4
LANES_ = 16

NW_ = 32
ROWS_ = BATCH_ * MAXLEN_
POS_W_ = MAXLEN_ // NW_
CHUNK_ = 32
HALF_ = POS_W_ // CHUNK_
NCHUNK_ = BATCH_ * HALF_
VECS_ = D_MODEL_ // LANES_


def _positional_encoding(maxlen, d_model):
    pos = np.arange(maxlen, dtype=np.float32)[:, None]
    i = np.arange(d_model, dtype=np.float32)[None, :]
    angle_rates = 1.0 / np.power(10000.0, (2.0 * np.floor(i / 2.0)) / np.float32(d_model))
    angle_rads = pos * angle_rates
    pe = np.zeros((maxlen, d_model), dtype=np.float32)
    pe[:, 0::2] = np.sin(angle_rads[:, 0::2])
    pe[:, 1::2] = np.cos(angle_rads[:, 1::2])
    return jnp.asarray(pe)


NBUF_ = 4
AHEAD_ = 2


def _chunk_idx(k):
    return k // BATCH_, k % BATCH_


def _emb_body(table_hbm, x_hbm, pe_hbm, out_hbm,
              idx_v, b0, b1, b2, b3, pe_v,
              g0, g1, g2, g3, o0, o1, o2, o3):
    wid = lax.axis_index("s") * 2 + lax.axis_index("c")
    pbase = wid * POS_W_
    bufs = (b0, b1, b2, b3)
    gsems = (g0, g1, g2, g3)
    osems = (o0, o1, o2, o3)

    pltpu.sync_copy(x_hbm.at[wid], idx_v)
    gcopy = [None] * NBUF_
    ocopy = [None] * NBUF_

    def gather(k):
        return pltpu.async_copy(
            table_hbm.at[idx_v.at[k]], bufs[k % NBUF_], gsems[k % NBUF_])

    for k in range(AHEAD_):
        gcopy[k] = gather(k)
    pltpu.sync_copy(pe_hbm.at[pl.ds(pbase, CHUNK_)], pe_v)

    for k in range(NCHUNK_):
        h, b = _chunk_idx(k)
        if k == BATCH_:
            pltpu.sync_copy(
                pe_hbm.at[pl.ds(pbase + CHUNK_, CHUNK_)], pe_v)
        buf = bufs[k % NBUF_]
        gcopy[k % NBUF_].wait()
        if k + AHEAD_ < NCHUNK_:
            nb = (k + AHEAD_) % NBUF_
            if ocopy[nb] is not None:
                ocopy[nb].wait()
                ocopy[nb] = None
            gcopy[nb] = gather(k + AHEAD_)

        def add_row(r, carry):
            for j in range(VECS_):
                sl = pl.ds(j * LANES_, LANES_)
                plsc.addupdate(buf.at[r, sl], pe_v[r, sl])
            return carry

        lax.fori_loop(0, CHUNK_, add_row, 0)
        obase = b * MAXLEN_ + pbase + h * CHUNK_
        ocopy[k % NBUF_] = pltpu.async_copy(
            buf, out_hbm.at[pl.ds(obase, CHUNK_)], osems[k % NBUF_])

    for oc in ocopy:
        if oc is not None:
            oc.wait()


def kernel(x, token_emb_table):
    idx = (x.reshape(BATCH_, NW_, HALF_, CHUNK_)
             .transpose(1, 2, 0, 3)
             .reshape(NW_, NCHUNK_, CHUNK_)
             .astype(jnp.int32))
    pe = _positional_encoding(MAXLEN_, D_MODEL_)
    mesh = plsc.VectorSubcoreMesh(core_axis_name="c", subcore_axis_name="s")
    out = pl.kernel(
        _emb_body,
        out_type=jax.ShapeDtypeStruct((ROWS_, D_MODEL_), jnp.float32),
        mesh=mesh,
        scratch_types=(
            [pltpu.VMEM((NCHUNK_, CHUNK_), jnp.int32)]
            + [pltpu.VMEM((CHUNK_, D_MODEL_), jnp.float32)] * NBUF_
            + [pltpu.VMEM((CHUNK_, D_MODEL_), jnp.float32)]
            + [pltpu.SemaphoreType.DMA] * (2 * NBUF_)
        ),
    )(token_emb_table, idx, pe)
    return out.reshape(BATCH_, MAXLEN_, D_MODEL_)


# speedup vs baseline: 1.4033x; 1.0209x over previous
import numpy as np
import jax
import jax.numpy as jnp
from jax import lax
from jax.experimental import pallas as pl
from jax.experimental.pallas import tpu as pltpu
from jax.experimental.pallas import tpu_sc as plsc

MAXLEN_ = 2048
D_MODEL_ = 768
BATCH_ = 4
LANES_ = 16

NW_ = 32
ROWS_ = BATCH_ * MAXLEN_
POS_W_ = MAXLEN_ // NW_
CHUNK_ = 32
HALF_ = POS_W_ // CHUNK_
NCHUNK_ = BATCH_ * HALF_
VECS_ = D_MODEL_ // LANES_


def _positional_encoding(maxlen, d_model):
    pos = np.arange(maxlen, dtype=np.float32)[:, None]
    i = np.arange(d_model, dtype=np.float32)[None, :]
    angle_rates = 1.0 / np.power(10000.0, (2.0 * np.floor(i / 2.0)) / np.float32(d_model))
    angle_rads = pos * angle_rates
    pe = np.zeros((maxlen, d_model), dtype=np.float32)
    pe[:, 0::2] = np.sin(angle_rads[:, 0::2])
    pe[:, 1::2] = np.cos(angle_rads[:, 1::2])
    return jnp.asarray(pe)


NBUF_ = 4
AHEAD_ = 2


def _chunk_idx(k):
    return k // BATCH_, k % BATCH_


def _emb_body(table_hbm, x_hbm, pe_hbm, out_hbm,
              idx_v, b0, b1, b2, b3, pe_v,
              g0, g1, g2, g3, o0, o1, o2, o3):
    wid = lax.axis_index("s") * 2 + lax.axis_index("c")
    pbase = wid * POS_W_
    bufs = (b0, b1, b2, b3)
    gsems = (g0, g1, g2, g3)
    osems = (o0, o1, o2, o3)

    for b in range(BATCH_):
        pltpu.sync_copy(x_hbm.at[b, pl.ds(pbase, POS_W_)], idx_v.at[b])
    gcopy = [None] * NBUF_
    ocopy = [None] * NBUF_

    def gather(k):
        h, b = _chunk_idx(k)
        return pltpu.async_copy(
            table_hbm.at[idx_v.at[b, pl.ds(h * CHUNK_, CHUNK_)]],
            bufs[k % NBUF_], gsems[k % NBUF_])

    for k in range(AHEAD_):
        gcopy[k] = gather(k)
    pltpu.sync_copy(pe_hbm.at[pl.ds(pbase, CHUNK_)], pe_v)

    for k in range(NCHUNK_):
        h, b = _chunk_idx(k)
        if k == BATCH_:
            pltpu.sync_copy(
                pe_hbm.at[pl.ds(pbase + CHUNK_, CHUNK_)], pe_v)
        buf = bufs[k % NBUF_]
        gcopy[k % NBUF_].wait()
        if k + AHEAD_ < NCHUNK_:
            nb = (k + AHEAD_) % NBUF_
            if ocopy[nb] is not None:
                ocopy[nb].wait()
                ocopy[nb] = None
            gcopy[nb] = gather(k + AHEAD_)

        def add_row(r, carry):
            for j in range(VECS_):
                sl = pl.ds(j * LANES_, LANES_)
                plsc.addupdate(buf.at[r, sl], pe_v[r, sl])
            return carry

        lax.fori_loop(0, CHUNK_, add_row, 0)
        obase = b * MAXLEN_ + pbase + h * CHUNK_
        ocopy[k % NBUF_] = pltpu.async_copy(
            buf, out_hbm.at[pl.ds(obase, CHUNK_)], osems[k % NBUF_])

    for oc in ocopy:
        if oc is not None:
            oc.wait()


def kernel(x, token_emb_table):
    pe = _positional_encoding(MAXLEN_, D_MODEL_)
    mesh = plsc.VectorSubcoreMesh(core_axis_name="c", subcore_axis_name="s")
    out = pl.kernel(
        _emb_body,
        out_type=jax.ShapeDtypeStruct((ROWS_, D_MODEL_), jnp.float32),
        mesh=mesh,
        scratch_types=(
            [pltpu.VMEM((BATCH_, POS_W_), jnp.int32)]
            + [pltpu.VMEM((CHUNK_, D_MODEL_), jnp.float32)] * NBUF_
            + [pltpu.VMEM((CHUNK_, D_MODEL_), jnp.float32)]
            + [pltpu.SemaphoreType.DMA] * (2 * NBUF_)
        ),
    )(token_emb_table, x.astype(jnp.int32), pe)
    return out.reshape(BATCH_, MAXLEN_, D_MODEL_)


# speedup vs baseline: 1.4443x; 1.0292x over previous
import numpy as np
import jax
import jax.numpy as jnp
from jax import lax
from jax.experimental import pallas as pl
from jax.experimental.pallas import tpu as pltpu
from jax.experimental.pallas import tpu_sc as plsc

MAXLEN_ = 2048
D_MODEL_ = 768
BATCH_ = 4
LANES_ = 16

NW_ = 32
ROWS_ = BATCH_ * MAXLEN_
POS_W_ = MAXLEN_ // NW_
CHUNK_ = 32
HALF_ = POS_W_ // CHUNK_
NCHUNK_ = BATCH_ * HALF_
VECS_ = D_MODEL_ // LANES_


def _positional_encoding(maxlen, d_model):
    pos = np.arange(maxlen, dtype=np.float32)[:, None]
    i = np.arange(d_model, dtype=np.float32)[None, :]
    angle_rates = 1.0 / np.power(10000.0, (2.0 * np.floor(i / 2.0)) / np.float32(d_model))
    angle_rads = pos * angle_rates
    pe = np.zeros((maxlen, d_model), dtype=np.float32)
    pe[:, 0::2] = np.sin(angle_rads[:, 0::2])
    pe[:, 1::2] = np.cos(angle_rads[:, 1::2])
    return jnp.asarray(pe)


NBUF_ = 4
AHEAD_ = 2


def _chunk_idx(k):
    return k // BATCH_, k % BATCH_


def _emb_body(table_hbm, x_hbm, pe_hbm, out_hbm,
              idx_v, b0, b1, b2, b3, pe_v,
              g0, g1, g2, g3, o0, o1, o2, o3):
    wid = lax.axis_index("s") * 2 + lax.axis_index("c")
    pbase = wid * POS_W_
    bufs = (b0, b1, b2, b3)
    gsems = (g0, g1, g2, g3)
    osems = (o0, o1, o2, o3)

    for b in range(BATCH_):
        pltpu.sync_copy(x_hbm.at[b, pl.ds(pbase, POS_W_)], idx_v.at[b])
    gcopy = [None] * NBUF_
    ocopy = [None] * NBUF_

    def gather(k):
        h, b = _chunk_idx(k)
        return pltpu.async_copy(
            table_hbm.at[idx_v.at[b, pl.ds(h * CHUNK_, CHUNK_)]],
            bufs[k % NBUF_], gsems[k % NBUF_])

    for k in range(AHEAD_):
        gcopy[k] = gather(k)
    pltpu.sync_copy(pe_hbm.at[pl.ds(pbase, CHUNK_)], pe_v)

    for k in range(NCHUNK_):
        h, b = _chunk_idx(k)
        if k == BATCH_:
            pltpu.sync_copy(
                pe_hbm.at[pl.ds(pbase + CHUNK_, CHUNK_)], pe_v)
        buf = bufs[k % NBUF_]
        gcopy[k % NBUF_].wait()
        if k + AHEAD_ < NCHUNK_:
            nb = (k + AHEAD_) % NBUF_
            if ocopy[nb] is not None:
                ocopy[nb].wait()
                ocopy[nb] = None
            gcopy[nb] = gather(k + AHEAD_)

        def add_rows(i, carry):
            r = i * 2
            for rr in (r, r + 1):
                for j in range(VECS_):
                    sl = pl.ds(j * LANES_, LANES_)
                    plsc.addupdate(buf.at[rr, sl], pe_v[rr, sl])
            return carry

        lax.fori_loop(0, CHUNK_ // 2, add_rows, 0)
        obase = b * MAXLEN_ + pbase + h * CHUNK_
        ocopy[k % NBUF_] = pltpu.async_copy(
            buf, out_hbm.at[pl.ds(obase, CHUNK_)], osems[k % NBUF_])

    for oc in ocopy:
        if oc is not None:
            oc.wait()


def kernel(x, token_emb_table):
    pe = _positional_encoding(MAXLEN_, D_MODEL_)
    mesh = plsc.VectorSubcoreMesh(core_axis_name="c", subcore_axis_name="s")
    out = pl.kernel(
        _emb_body,
        out_type=jax.ShapeDtypeStruct((ROWS_, D_MODEL_), jnp.float32),
        mesh=mesh,
        scratch_types=(
            [pltpu.VMEM((BATCH_, POS_W_), jnp.int32)]
            + [pltpu.VMEM((CHUNK_, D_MODEL_), jnp.float32)] * NBUF_
            + [pltpu.VMEM((CHUNK_, D_MODEL_), jnp.float32)]
            + [pltpu.SemaphoreType.DMA] * (2 * NBUF_)
        ),
    )(token_emb_table, x.astype(jnp.int32), pe)
    return out.reshape(BATCH_, MAXLEN_, D_MODEL_)
